# baseline (speedup 1.0000x reference)
import jax
import jax.numpy as jnp
from jax import lax
from jax.experimental import pallas as pl
from jax.experimental.pallas import tpu as pltpu
from jax.experimental.pallas import tpu_sc as plsc

D = 128
EPS = 1e-12
NC = 2
NS = 16
NW = NC * NS
L = 16


def _body(idx_hbm, emb_hbm, pos_hbm, gamma_hbm, beta_hbm, out_hbm,
          idx_v, rows_v, pos_v, gb_v, mean_v, scale_v, sem):
    rows_per_w = rows_v.shape[0]
    n_idx_chunks = idx_v.shape[0]
    idx_chunk = idx_v.shape[1]
    groups = rows_per_w // L
    seq_len = pos_hbm.shape[0]

    wid = lax.axis_index("s") * NC + lax.axis_index("c")
    base = wid * rows_per_w
    s0 = lax.rem(base, seq_len)

    for k in range(n_idx_chunks):
        pltpu.sync_copy(idx_hbm.at[pl.ds(base + k * idx_chunk, idx_chunk)],
                        idx_v.at[k])
    copies = [pltpu.async_copy(emb_hbm.at[idx_v.at[k]],
                               rows_v.at[pl.ds(k * idx_chunk, idx_chunk)], sem)
              for k in range(n_idx_chunks)]
    pltpu.sync_copy(pos_hbm.at[pl.ds(s0, rows_per_w)], pos_v)
    pltpu.sync_copy(gamma_hbm, gb_v.at[0])
    pltpu.sync_copy(beta_hbm, gb_v.at[1])
    for c in copies:
        c.wait()

    lane = jnp.arange(L, dtype=jnp.int32)
    inv_d = jnp.float32(1.0 / D)

    def group_body(g, _):
        ridx = g * L + lane

        def d_body(dd, carry):
            s, ss = carry
            dcol = jnp.full((L,), dd, jnp.int32)
            v = (plsc.load_gather(rows_v, [ridx, dcol])
                 + plsc.load_gather(pos_v, [ridx, dcol]))
            plsc.store_scatter(rows_v, [ridx, dcol], v)
            return (s + v, ss + v * v)

        zero = jnp.zeros((L,), jnp.float32)
        s, ss = lax.fori_loop(0, D, d_body, (zero, zero))
        mean = s * inv_d
        var = ss * inv_d - mean * mean
        x = var + jnp.float32(EPS)
        i = lax.bitcast_convert_type(x, jnp.int32)
        i = jnp.int32(0x5F3759DF) - (i >> 1)
        y = lax.bitcast_convert_type(i, jnp.float32)
        for _ in range(3):
            y = y * (jnp.float32(1.5) - jnp.float32(0.5) * x * y * y)
        mean_v[pl.ds(g * L, L)] = mean
        scale_v[pl.ds(g * L, L)] = y
        return 0

    lax.fori_loop(0, groups, group_body, 0)

    def row_body(r, _):
        m = mean_v[pl.ds(r, L)][0]
        sc = scale_v[pl.ds(r, L)][0]
        for j in range(D // L):
            sl = pl.ds(j * L, L)
            v = rows_v[r, sl]
            rows_v[r, sl] = (v - m) * sc * gb_v[0, sl] + gb_v[1, sl]
        return 0

    lax.fori_loop(0, rows_per_w, row_body, 0)
    pltpu.sync_copy(rows_v, out_hbm.at[pl.ds(base, rows_per_w)])


def kernel(inputs, emb_table, pos_table, gamma, beta):
    b, s = inputs.shape
    n = b * s
    rows_per_w = n // NW
    idx_chunk = min(rows_per_w, 128)
    idx = inputs.reshape(n).astype(jnp.int32)

    mesh = plsc.VectorSubcoreMesh(core_axis_name="c", subcore_axis_name="s")
    out = pl.kernel(
        _body,
        mesh=mesh,
        compiler_params=pltpu.CompilerParams(needs_layout_passes=False),
        out_type=jax.ShapeDtypeStruct((n, D), jnp.float32),
        scratch_types=[
            pltpu.VMEM((rows_per_w // idx_chunk, idx_chunk), jnp.int32),
            pltpu.VMEM((rows_per_w, D), jnp.float32),
            pltpu.VMEM((rows_per_w, D), jnp.float32),
            pltpu.VMEM((2, D), jnp.float32),
            pltpu.VMEM((rows_per_w + 16,), jnp.float32),
            pltpu.VMEM((rows_per_w + 16,), jnp.float32),
            pltpu.SemaphoreType.DMA,
        ],
    )(idx, emb_table, pos_table, gamma, beta)
    return out.reshape(b, s, D)
